# baseline (speedup 1.0000x reference)
import functools

import jax
import jax.numpy as jnp
import numpy as np
from jax import lax
from jax.experimental import pallas as pl
from jax.experimental.pallas import tpu as pltpu
from jax.experimental.pallas import tpu_sc as plsc

B, N, F_IN, H, K = 2, 4096, 64, 64, 16
BN = B * N
BNK = BN * K


TR = 32
NSLAB = N // 128
TOPL = 6


def _knn_body(boff, xyz_ref, xyzt_ref, idx_ref):
    xyz_tile = xyz_ref[...]
    xc = xyzt_ref[...]
    xt2 = xyz_tile * xyz_tile
    sqr = xt2[:, 0:1] + xt2[:, 1:2] + xt2[:, 2:3]
    xc2 = xc * xc
    sqc = xc2[0:1, :] + xc2[1:2, :] + xc2[2:3, :]
    xr0 = xyz_tile[:, 0:1]
    xr1 = xyz_tile[:, 1:2]
    xr2 = xyz_tile[:, 2:3]
    lane = jax.lax.broadcasted_iota(jnp.int32, (TR, 128), 1)
    inf = jnp.float32(3.0e38)
    big_i = jnp.int32(1 << 30)
    m = [jnp.full((TR, 128), inf, jnp.float32) for _ in range(TOPL)]
    mi = [jnp.zeros((TR, 128), jnp.int32) for _ in range(TOPL)]
    for j in range(NSLAB):
        sl = slice(j * 128, (j + 1) * 128)
        dot = xr0 * xc[0:1, sl] + xr1 * xc[1:2, sl] + xr2 * xc[2:3, sl]
        v = (sqr + sqc[:, sl]) - 2.0 * dot
        vi = lane + (j * 128)
        for t in range(TOPL):
            c = v < m[t]
            nm = jnp.where(c, v, m[t])
            ni = jnp.where(c, vi, mi[t])
            v = jnp.where(c, m[t], v)
            vi = jnp.where(c, mi[t], vi)
            m[t], mi[t] = nm, ni
    for k in range(K):
        mn = jnp.min(m[0], axis=1, keepdims=True)
        eq = m[0] == mn
        am = jnp.min(jnp.where(eq, mi[0], big_i),
                     axis=1, keepdims=True)
        idx_ref[:, k:k + 1] = am + boff
        win = eq & (mi[0] == am)
        for t in range(TOPL - 1):
            m[t] = jnp.where(win, m[t + 1], m[t])
            mi[t] = jnp.where(win, mi[t + 1], mi[t])
        m[TOPL - 1] = jnp.where(win, inf, m[TOPL - 1])


def _knn_topk(xyz_b, boff):
    xyzt = jnp.transpose(xyz_b, (1, 0))
    return pl.pallas_call(
        functools.partial(_knn_body, boff),
        grid=(N // TR,),
        in_specs=[
            pl.BlockSpec((TR, 3), lambda i: (i, 0)),
            pl.BlockSpec((3, N), lambda i: (0, 0)),
        ],
        out_specs=pl.BlockSpec((TR, K), lambda i: (i, 0)),
        out_shape=jax.ShapeDtypeStruct((N, K), jnp.int32),
        compiler_params=pltpu.CompilerParams(
            dimension_semantics=("parallel",)),
    )(xyz_b, xyzt)


def _proj_body(f_ref, xyz_ref, wk_ref, bk_ref, wq_ref, wkk_ref, wv_ref,
               q_ref, tbl_ref):
    x = jnp.dot(f_ref[...], wk_ref[...], preferred_element_type=jnp.float32)
    x = x + bk_ref[...]
    q_ref[...] = jnp.dot(x, wq_ref[...], preferred_element_type=jnp.float32)
    xk = jnp.dot(x, wkk_ref[...], preferred_element_type=jnp.float32)
    xv = jnp.dot(x, wv_ref[...], preferred_element_type=jnp.float32)
    half = jnp.int32(0x8000)
    him = jnp.int32(-65536)
    lom = jnp.int32(0xFFFF)
    hi = (pltpu.bitcast(xk, jnp.int32) + half) & him
    lo = ((pltpu.bitcast(xv, jnp.int32) + half) >> 16) & lom
    tbl_ref[:, 0:H] = pltpu.bitcast(hi | lo, jnp.float32)
    tbl_ref[:, H:H + 3] = xyz_ref[...]


SC_CHUNK = 512


def _sc_gather(tbl, idxg):
    nidx = idxg.shape[0]
    info = plsc.get_sparse_core_info()
    nw = info.num_cores * info.num_subcores
    b_per_w = nidx // nw
    mesh = plsc.VectorSubcoreMesh(core_axis_name="c", subcore_axis_name="s")

    @functools.partial(
        pl.kernel, mesh=mesh,
        out_type=jax.ShapeDtypeStruct((nidx, 128), jnp.float32),
        scratch_types=[pltpu.VMEM((SC_CHUNK,), jnp.int32),
                       pltpu.VMEM((SC_CHUNK, 128), jnp.float32)],
    )
    def k(tbl_hbm, idx_hbm, out_hbm, idx_v, rows_v):
        wid = lax.axis_index("s") * info.num_cores + lax.axis_index("c")
        base = wid * b_per_w

        @pl.loop(0, b_per_w, step=SC_CHUNK)
        def _(off):
            pltpu.sync_copy(idx_hbm.at[pl.ds(base + off, SC_CHUNK)], idx_v)
            pltpu.sync_copy(tbl_hbm.at[idx_v], rows_v)
            pltpu.sync_copy(rows_v, out_hbm.at[pl.ds(base + off, SC_CHUNK)])

    return k(tbl, idxg)


def _mlp_body(T, q_ref, gat_ref, xyz_ref, feat_ref,
              pw1_ref, pb1_ref, pw2_ref, pb2_ref,
              aw1_ref, ab1_ref, aw2_ref, ab2_ref,
              wagg_ref, bagg_ref, att_ref, res_ref):
    TK = T * K
    gat = gat_ref[...]
    xyz = xyz_ref[...]
    kxyz = gat[:, H:H + 3]
    rel = (xyz[:, None, :] - kxyz.reshape(T, K, 3)).reshape(TK, 3)
    pw1 = pw1_ref[...]
    pos1 = (rel[:, 0:1] * pw1[0:1, :] + rel[:, 1:2] * pw1[1:2, :]
            + rel[:, 2:3] * pw1[2:3, :] + pb1_ref[...])
    posr = jnp.maximum(pos1, 0.0)
    pos_enc = jnp.dot(posr, pw2_ref[...], preferred_element_type=jnp.float32)
    pos_enc = pos_enc + pb2_ref[...]
    q = q_ref[...]
    bits = pltpu.bitcast(gat[:, 0:H], jnp.int32)
    kg = pltpu.bitcast(bits & jnp.int32(-65536), jnp.float32)
    vg = pltpu.bitcast(bits << 16, jnp.float32)
    u = (q[:, None, :] - kg.reshape(T, K, H)
         + pos_enc.reshape(T, K, H)).reshape(TK, H)
    a1 = jnp.dot(u, aw1_ref[...], preferred_element_type=jnp.float32)
    a1 = jnp.maximum(a1 + ab1_ref[...], 0.0)
    att = jnp.dot(a1, aw2_ref[...], preferred_element_type=jnp.float32)
    att = att + ab2_ref[...]
    s3 = (att * (1.0 / np.sqrt(H))).reshape(T, K, H)
    m = jnp.max(s3, axis=1, keepdims=True)
    e = jnp.exp(s3 - m)
    att_sm = e / jnp.sum(e, axis=1, keepdims=True)
    att_ref[...] = att_sm.reshape(TK, H)
    w = att_sm * (vg.reshape(T, K, H) + pos_enc.reshape(T, K, H))
    r = jnp.sum(w, axis=1)
    res = jnp.dot(r, wagg_ref[...], preferred_element_type=jnp.float32)
    res_ref[...] = res + bagg_ref[...] + feat_ref[...]


def kernel(xyz, features, W_kernel, b_kernel, W_agg, b_agg, W_q, W_k, W_v,
           pos_W1, pos_b1, pos_W2, pos_b2, att_W1, att_b1, att_W2, att_b2):
    f2 = features.reshape(BN, F_IN)

    TP = 2048
    q, tbl = pl.pallas_call(
        _proj_body,
        grid=(BN // TP,),
        in_specs=[
            pl.BlockSpec((TP, F_IN), lambda i: (i, 0)),
            pl.BlockSpec((TP, 3), lambda i: (i, 0)),
            pl.BlockSpec((F_IN, H), lambda i: (0, 0)),
            pl.BlockSpec((1, H), lambda i: (0, 0)),
            pl.BlockSpec((H, H), lambda i: (0, 0)),
            pl.BlockSpec((H, H), lambda i: (0, 0)),
            pl.BlockSpec((H, H), lambda i: (0, 0)),
        ],
        out_specs=[
            pl.BlockSpec((TP, H), lambda i: (i, 0)),
            pl.BlockSpec((TP, 128), lambda i: (i, 0)),
        ],
        out_shape=[jax.ShapeDtypeStruct((BN, H), jnp.float32),
                   jax.ShapeDtypeStruct((BN, 128), jnp.float32)],
        compiler_params=pltpu.CompilerParams(
            dimension_semantics=("parallel",)),
    )(f2, xyz.reshape(BN, 3), W_kernel, b_kernel.reshape(1, H), W_q, W_k, W_v)

    T = 512
    xyz2 = xyz.reshape(BN, 3)
    idxs = [_knn_topk(xyz[b], b * N) for b in range(B)]
    gats = [_sc_gather(tbl, idxs[b].reshape(N * K)) for b in range(B)]

    def _mlp_batch(b, gat_b):
        nblk = N // T
        return pl.pallas_call(
            functools.partial(_mlp_body, T),
            grid=(nblk,),
            in_specs=[
                pl.BlockSpec((T, H), lambda i: (i + b * nblk, 0)),
                pl.BlockSpec((T * K, 128), lambda i: (i, 0)),
                pl.BlockSpec((T, 3), lambda i: (i + b * nblk, 0)),
                pl.BlockSpec((T, F_IN), lambda i: (i + b * nblk, 0)),
                pl.BlockSpec((3, H), lambda i: (0, 0)),
                pl.BlockSpec((1, H), lambda i: (0, 0)),
                pl.BlockSpec((H, H), lambda i: (0, 0)),
                pl.BlockSpec((1, H), lambda i: (0, 0)),
                pl.BlockSpec((H, H), lambda i: (0, 0)),
                pl.BlockSpec((1, H), lambda i: (0, 0)),
                pl.BlockSpec((H, H), lambda i: (0, 0)),
                pl.BlockSpec((1, H), lambda i: (0, 0)),
                pl.BlockSpec((H, H), lambda i: (0, 0)),
                pl.BlockSpec((1, F_IN), lambda i: (0, 0)),
            ],
            out_specs=[
                pl.BlockSpec((T * K, H), lambda i: (i, 0)),
                pl.BlockSpec((T, H), lambda i: (i, 0)),
            ],
            out_shape=[
                jax.ShapeDtypeStruct((N * K, H), jnp.float32),
                jax.ShapeDtypeStruct((N, F_IN), jnp.float32),
            ],
            compiler_params=pltpu.CompilerParams(
                dimension_semantics=("parallel",)),
        )(q, gat_b, xyz2, f2,
          pos_W1, pos_b1.reshape(1, H), pos_W2, pos_b2.reshape(1, H),
          att_W1, att_b1.reshape(1, H), att_W2, att_b2.reshape(1, H),
          W_agg, b_agg.reshape(1, F_IN))

    outs = [_mlp_batch(b, gats[b]) for b in range(B)]
    res = jnp.stack([o[1] for o in outs])
    att = jnp.stack([o[0].reshape(N, K, H) for o in outs])
    return (res, att)


# speedup vs baseline: 2.1682x; 2.1682x over previous
import functools

import jax
import jax.numpy as jnp
import numpy as np
from jax import lax
from jax.experimental import pallas as pl
from jax.experimental.pallas import tpu as pltpu
from jax.experimental.pallas import tpu_sc as plsc

B, N, F_IN, H, K = 2, 4096, 64, 64, 16
BN = B * N
BNK = BN * K


TR = 128
NSLAB = N // 128


def _knn_body(boff, xyz_ref, xyzt_ref, idx_ref, d2_ref):
    xyz_tile = xyz_ref[...]
    xc = xyzt_ref[...]
    xt2 = xyz_tile * xyz_tile
    sqr = xt2[:, 0:1] + xt2[:, 1:2] + xt2[:, 2:3]
    xc2 = xc * xc
    sqc = xc2[0:1, :] + xc2[1:2, :] + xc2[2:3, :]
    xr0 = xyz_tile[:, 0:1]
    xr1 = xyz_tile[:, 1:2]
    xr2 = xyz_tile[:, 2:3]
    for j in range(NSLAB):
        sl = slice(j * 128, (j + 1) * 128)
        dot = xr0 * xc[0:1, sl] + xr1 * xc[1:2, sl] + xr2 * xc[2:3, sl]
        d2_ref[:, sl] = (sqr + sqc[:, sl]) - 2.0 * dot
    lane = jax.lax.broadcasted_iota(jnp.int32, (TR, 128), 1)
    inf = jnp.float32(3.0e38)
    big_i = jnp.int32(1 << 30)
    prev = jnp.full((TR, 1), -inf, jnp.float32)
    for k in range(K):
        mnl = jnp.full((TR, 128), inf, jnp.float32)
        aml = jnp.zeros((TR, 128), jnp.int32)
        for j in range(NSLAB):
            v = d2_ref[:, j * 128:(j + 1) * 128]
            upd = (v > prev) & (v < mnl)
            mnl = jnp.where(upd, v, mnl)
            aml = jnp.where(upd, lane + (j * 128), aml)
        mn = jnp.min(mnl, axis=1, keepdims=True)
        am = jnp.min(jnp.where(mnl == mn, aml, big_i),
                     axis=1, keepdims=True)
        idx_ref[:, k:k + 1] = am + boff
        prev = mn


def _knn_topk(xyz_b, boff):
    xyzt = jnp.transpose(xyz_b, (1, 0))
    return pl.pallas_call(
        functools.partial(_knn_body, boff),
        grid=(N // TR,),
        in_specs=[
            pl.BlockSpec((TR, 3), lambda i: (i, 0)),
            pl.BlockSpec((3, N), lambda i: (0, 0)),
        ],
        out_specs=pl.BlockSpec((TR, K), lambda i: (i, 0)),
        out_shape=jax.ShapeDtypeStruct((N, K), jnp.int32),
        scratch_shapes=[pltpu.VMEM((TR, N), jnp.float32)],
        compiler_params=pltpu.CompilerParams(
            dimension_semantics=("parallel",)),
    )(xyz_b, xyzt)


def _proj_body(f_ref, xyz_ref, wk_ref, bk_ref, wq_ref, wkk_ref, wv_ref,
               q_ref, tbl_ref):
    x = jnp.dot(f_ref[...], wk_ref[...], preferred_element_type=jnp.float32)
    x = x + bk_ref[...]
    q_ref[...] = jnp.dot(x, wq_ref[...], preferred_element_type=jnp.float32)
    xk = jnp.dot(x, wkk_ref[...], preferred_element_type=jnp.float32)
    xv = jnp.dot(x, wv_ref[...], preferred_element_type=jnp.float32)
    half = jnp.int32(0x8000)
    him = jnp.int32(-65536)
    lom = jnp.int32(0xFFFF)
    hi = (pltpu.bitcast(xk, jnp.int32) + half) & him
    lo = ((pltpu.bitcast(xv, jnp.int32) + half) >> 16) & lom
    tbl_ref[:, 0:H] = pltpu.bitcast(hi | lo, jnp.float32)
    tbl_ref[:, H:H + 3] = xyz_ref[...]


SC_CHUNK = 512


def _sc_gather(tbl, idxg):
    nidx = idxg.shape[0]
    info = plsc.get_sparse_core_info()
    nw = info.num_cores * info.num_subcores
    b_per_w = nidx // nw
    mesh = plsc.VectorSubcoreMesh(core_axis_name="c", subcore_axis_name="s")

    @functools.partial(
        pl.kernel, mesh=mesh,
        out_type=jax.ShapeDtypeStruct((nidx, 128), jnp.float32),
        scratch_types=[pltpu.VMEM((SC_CHUNK,), jnp.int32),
                       pltpu.VMEM((SC_CHUNK, 128), jnp.float32)],
    )
    def k(tbl_hbm, idx_hbm, out_hbm, idx_v, rows_v):
        wid = lax.axis_index("s") * info.num_cores + lax.axis_index("c")
        base = wid * b_per_w

        @pl.loop(0, b_per_w, step=SC_CHUNK)
        def _(off):
            pltpu.sync_copy(idx_hbm.at[pl.ds(base + off, SC_CHUNK)], idx_v)
            pltpu.sync_copy(tbl_hbm.at[idx_v], rows_v)
            pltpu.sync_copy(rows_v, out_hbm.at[pl.ds(base + off, SC_CHUNK)])

    return k(tbl, idxg)


def _mlp_body(T, q_ref, gat_ref, xyz_ref, feat_ref,
              pw1_ref, pb1_ref, pw2_ref, pb2_ref,
              aw1_ref, ab1_ref, aw2_ref, ab2_ref,
              wagg_ref, bagg_ref, att_ref, res_ref):
    TK = T * K
    gat = gat_ref[...]
    xyz = xyz_ref[...]
    kxyz = gat[:, H:H + 3]
    rel = (xyz[:, None, :] - kxyz.reshape(T, K, 3)).reshape(TK, 3)
    pw1 = pw1_ref[...]
    pos1 = (rel[:, 0:1] * pw1[0:1, :] + rel[:, 1:2] * pw1[1:2, :]
            + rel[:, 2:3] * pw1[2:3, :] + pb1_ref[...])
    posr = jnp.maximum(pos1, 0.0)
    pos_enc = jnp.dot(posr, pw2_ref[...], preferred_element_type=jnp.float32)
    pos_enc = pos_enc + pb2_ref[...]
    q = q_ref[...]
    bits = pltpu.bitcast(gat[:, 0:H], jnp.int32)
    kg = pltpu.bitcast(bits & jnp.int32(-65536), jnp.float32)
    vg = pltpu.bitcast(bits << 16, jnp.float32)
    u = (q[:, None, :] - kg.reshape(T, K, H)
         + pos_enc.reshape(T, K, H)).reshape(TK, H)
    a1 = jnp.dot(u, aw1_ref[...], preferred_element_type=jnp.float32)
    a1 = jnp.maximum(a1 + ab1_ref[...], 0.0)
    att = jnp.dot(a1, aw2_ref[...], preferred_element_type=jnp.float32)
    att = att + ab2_ref[...]
    s3 = (att * (1.0 / np.sqrt(H))).reshape(T, K, H)
    m = jnp.max(s3, axis=1, keepdims=True)
    e = jnp.exp(s3 - m)
    att_sm = e / jnp.sum(e, axis=1, keepdims=True)
    att_ref[...] = att_sm.reshape(TK, H)
    w = att_sm * (vg.reshape(T, K, H) + pos_enc.reshape(T, K, H))
    r = jnp.sum(w, axis=1)
    res = jnp.dot(r, wagg_ref[...], preferred_element_type=jnp.float32)
    res_ref[...] = res + bagg_ref[...] + feat_ref[...]


def kernel(xyz, features, W_kernel, b_kernel, W_agg, b_agg, W_q, W_k, W_v,
           pos_W1, pos_b1, pos_W2, pos_b2, att_W1, att_b1, att_W2, att_b2):
    f2 = features.reshape(BN, F_IN)

    TP = 2048
    q, tbl = pl.pallas_call(
        _proj_body,
        grid=(BN // TP,),
        in_specs=[
            pl.BlockSpec((TP, F_IN), lambda i: (i, 0)),
            pl.BlockSpec((TP, 3), lambda i: (i, 0)),
            pl.BlockSpec((F_IN, H), lambda i: (0, 0)),
            pl.BlockSpec((1, H), lambda i: (0, 0)),
            pl.BlockSpec((H, H), lambda i: (0, 0)),
            pl.BlockSpec((H, H), lambda i: (0, 0)),
            pl.BlockSpec((H, H), lambda i: (0, 0)),
        ],
        out_specs=[
            pl.BlockSpec((TP, H), lambda i: (i, 0)),
            pl.BlockSpec((TP, 128), lambda i: (i, 0)),
        ],
        out_shape=[jax.ShapeDtypeStruct((BN, H), jnp.float32),
                   jax.ShapeDtypeStruct((BN, 128), jnp.float32)],
        compiler_params=pltpu.CompilerParams(
            dimension_semantics=("parallel",)),
    )(f2, xyz.reshape(BN, 3), W_kernel, b_kernel.reshape(1, H), W_q, W_k, W_v)

    T = 512
    xyz2 = xyz.reshape(BN, 3)
    idxs = [_knn_topk(xyz[b], b * N) for b in range(B)]
    gats = [_sc_gather(tbl, idxs[b].reshape(N * K)) for b in range(B)]

    def _mlp_batch(b, gat_b):
        nblk = N // T
        return pl.pallas_call(
            functools.partial(_mlp_body, T),
            grid=(nblk,),
            in_specs=[
                pl.BlockSpec((T, H), lambda i: (i + b * nblk, 0)),
                pl.BlockSpec((T * K, 128), lambda i: (i, 0)),
                pl.BlockSpec((T, 3), lambda i: (i + b * nblk, 0)),
                pl.BlockSpec((T, F_IN), lambda i: (i + b * nblk, 0)),
                pl.BlockSpec((3, H), lambda i: (0, 0)),
                pl.BlockSpec((1, H), lambda i: (0, 0)),
                pl.BlockSpec((H, H), lambda i: (0, 0)),
                pl.BlockSpec((1, H), lambda i: (0, 0)),
                pl.BlockSpec((H, H), lambda i: (0, 0)),
                pl.BlockSpec((1, H), lambda i: (0, 0)),
                pl.BlockSpec((H, H), lambda i: (0, 0)),
                pl.BlockSpec((1, H), lambda i: (0, 0)),
                pl.BlockSpec((H, H), lambda i: (0, 0)),
                pl.BlockSpec((1, F_IN), lambda i: (0, 0)),
            ],
            out_specs=[
                pl.BlockSpec((T * K, H), lambda i: (i, 0)),
                pl.BlockSpec((T, H), lambda i: (i, 0)),
            ],
            out_shape=[
                jax.ShapeDtypeStruct((N * K, H), jnp.float32),
                jax.ShapeDtypeStruct((N, F_IN), jnp.float32),
            ],
            compiler_params=pltpu.CompilerParams(
                dimension_semantics=("parallel",)),
        )(q, gat_b, xyz2, f2,
          pos_W1, pos_b1.reshape(1, H), pos_W2, pos_b2.reshape(1, H),
          att_W1, att_b1.reshape(1, H), att_W2, att_b2.reshape(1, H),
          W_agg, b_agg.reshape(1, F_IN))

    outs = [_mlp_batch(b, gats[b]) for b in range(B)]
    res = jnp.stack([o[1] for o in outs])
    att = jnp.stack([o[0].reshape(N, K, H) for o in outs])
    return (res, att)


# speedup vs baseline: 2.2360x; 1.0313x over previous
import functools

import jax
import jax.numpy as jnp
import numpy as np
from jax import lax
from jax.experimental import pallas as pl
from jax.experimental.pallas import tpu as pltpu
from jax.experimental.pallas import tpu_sc as plsc

B, N, F_IN, H, K = 2, 4096, 64, 64, 16
BN = B * N
BNK = BN * K


TR = 256
NSLAB = N // 128


def _knn_body(boff, xyz_ref, xyzt_ref, idx_ref, d2_ref):
    xyz_tile = xyz_ref[...]
    xc = xyzt_ref[...]
    xt2 = xyz_tile * xyz_tile
    sqr = xt2[:, 0:1] + xt2[:, 1:2] + xt2[:, 2:3]
    xc2 = xc * xc
    sqc = xc2[0:1, :] + xc2[1:2, :] + xc2[2:3, :]
    xr0 = xyz_tile[:, 0:1]
    xr1 = xyz_tile[:, 1:2]
    xr2 = xyz_tile[:, 2:3]
    for j in range(NSLAB):
        sl = slice(j * 128, (j + 1) * 128)
        dot = xr0 * xc[0:1, sl] + xr1 * xc[1:2, sl] + xr2 * xc[2:3, sl]
        d2_ref[:, sl] = (sqr + sqc[:, sl]) - 2.0 * dot
    lane = jax.lax.broadcasted_iota(jnp.int32, (TR, 128), 1)
    inf = jnp.float32(3.0e38)
    big_i = jnp.int32(1 << 30)
    prev = jnp.full((TR, 1), -inf, jnp.float32)
    for k in range(K):
        mnl = jnp.full((TR, 128), inf, jnp.float32)
        aml = jnp.zeros((TR, 128), jnp.int32)
        for j in range(NSLAB):
            v = d2_ref[:, j * 128:(j + 1) * 128]
            upd = (v > prev) & (v < mnl)
            mnl = jnp.where(upd, v, mnl)
            aml = jnp.where(upd, lane + (j * 128), aml)
        mn = jnp.min(mnl, axis=1, keepdims=True)
        am = jnp.min(jnp.where(mnl == mn, aml, big_i),
                     axis=1, keepdims=True)
        idx_ref[:, k:k + 1] = am + boff
        prev = mn


def _knn_topk(xyz_b, boff):
    xyzt = jnp.transpose(xyz_b, (1, 0))
    return pl.pallas_call(
        functools.partial(_knn_body, boff),
        grid=(N // TR,),
        in_specs=[
            pl.BlockSpec((TR, 3), lambda i: (i, 0)),
            pl.BlockSpec((3, N), lambda i: (0, 0)),
        ],
        out_specs=pl.BlockSpec((TR, K), lambda i: (i, 0)),
        out_shape=jax.ShapeDtypeStruct((N, K), jnp.int32),
        scratch_shapes=[pltpu.VMEM((TR, N), jnp.float32)],
        compiler_params=pltpu.CompilerParams(
            dimension_semantics=("parallel",)),
    )(xyz_b, xyzt)


def _proj_body(f_ref, xyz_ref, wk_ref, bk_ref, wq_ref, wkk_ref, wv_ref,
               q_ref, tbl_ref):
    x = jnp.dot(f_ref[...], wk_ref[...], preferred_element_type=jnp.float32)
    x = x + bk_ref[...]
    q_ref[...] = jnp.dot(x, wq_ref[...], preferred_element_type=jnp.float32)
    xk = jnp.dot(x, wkk_ref[...], preferred_element_type=jnp.float32)
    xv = jnp.dot(x, wv_ref[...], preferred_element_type=jnp.float32)
    half = jnp.int32(0x8000)
    him = jnp.int32(-65536)
    lom = jnp.int32(0xFFFF)
    hi = (pltpu.bitcast(xk, jnp.int32) + half) & him
    lo = ((pltpu.bitcast(xv, jnp.int32) + half) >> 16) & lom
    tbl_ref[:, 0:H] = pltpu.bitcast(hi | lo, jnp.float32)
    tbl_ref[:, H:H + 3] = xyz_ref[...]


SC_CHUNK = 512


def _sc_gather(tbl, idxg):
    nidx = idxg.shape[0]
    info = plsc.get_sparse_core_info()
    nw = info.num_cores * info.num_subcores
    b_per_w = nidx // nw
    mesh = plsc.VectorSubcoreMesh(core_axis_name="c", subcore_axis_name="s")

    @functools.partial(
        pl.kernel, mesh=mesh,
        out_type=jax.ShapeDtypeStruct((nidx, 128), jnp.float32),
        scratch_types=[pltpu.VMEM((SC_CHUNK,), jnp.int32),
                       pltpu.VMEM((SC_CHUNK, 128), jnp.float32)],
    )
    def k(tbl_hbm, idx_hbm, out_hbm, idx_v, rows_v):
        wid = lax.axis_index("s") * info.num_cores + lax.axis_index("c")
        base = wid * b_per_w

        @pl.loop(0, b_per_w, step=SC_CHUNK)
        def _(off):
            pltpu.sync_copy(idx_hbm.at[pl.ds(base + off, SC_CHUNK)], idx_v)
            pltpu.sync_copy(tbl_hbm.at[idx_v], rows_v)
            pltpu.sync_copy(rows_v, out_hbm.at[pl.ds(base + off, SC_CHUNK)])

    return k(tbl, idxg)


def _mlp_body(T, q_ref, gat_ref, xyz_ref, feat_ref,
              pw1_ref, pb1_ref, pw2_ref, pb2_ref,
              aw1_ref, ab1_ref, aw2_ref, ab2_ref,
              wagg_ref, bagg_ref, att_ref, res_ref):
    TK = T * K
    gat = gat_ref[...]
    xyz = xyz_ref[...]
    kxyz = gat[:, H:H + 3]
    rel = (xyz[:, None, :] - kxyz.reshape(T, K, 3)).reshape(TK, 3)
    pw1 = pw1_ref[...]
    pos1 = (rel[:, 0:1] * pw1[0:1, :] + rel[:, 1:2] * pw1[1:2, :]
            + rel[:, 2:3] * pw1[2:3, :] + pb1_ref[...])
    posr = jnp.maximum(pos1, 0.0)
    pos_enc = jnp.dot(posr, pw2_ref[...], preferred_element_type=jnp.float32)
    pos_enc = pos_enc + pb2_ref[...]
    q = q_ref[...]
    bits = pltpu.bitcast(gat[:, 0:H], jnp.int32)
    kg = pltpu.bitcast(bits & jnp.int32(-65536), jnp.float32)
    vg = pltpu.bitcast(bits << 16, jnp.float32)
    u = (q[:, None, :] - kg.reshape(T, K, H)
         + pos_enc.reshape(T, K, H)).reshape(TK, H)
    a1 = jnp.dot(u, aw1_ref[...], preferred_element_type=jnp.float32)
    a1 = jnp.maximum(a1 + ab1_ref[...], 0.0)
    att = jnp.dot(a1, aw2_ref[...], preferred_element_type=jnp.float32)
    att = att + ab2_ref[...]
    s3 = (att * (1.0 / np.sqrt(H))).reshape(T, K, H)
    m = jnp.max(s3, axis=1, keepdims=True)
    e = jnp.exp(s3 - m)
    att_sm = e / jnp.sum(e, axis=1, keepdims=True)
    att_ref[...] = att_sm.reshape(TK, H)
    w = att_sm * (vg.reshape(T, K, H) + pos_enc.reshape(T, K, H))
    r = jnp.sum(w, axis=1)
    res = jnp.dot(r, wagg_ref[...], preferred_element_type=jnp.float32)
    res_ref[...] = res + bagg_ref[...] + feat_ref[...]


def kernel(xyz, features, W_kernel, b_kernel, W_agg, b_agg, W_q, W_k, W_v,
           pos_W1, pos_b1, pos_W2, pos_b2, att_W1, att_b1, att_W2, att_b2):
    f2 = features.reshape(BN, F_IN)

    TP = 2048
    q, tbl = pl.pallas_call(
        _proj_body,
        grid=(BN // TP,),
        in_specs=[
            pl.BlockSpec((TP, F_IN), lambda i: (i, 0)),
            pl.BlockSpec((TP, 3), lambda i: (i, 0)),
            pl.BlockSpec((F_IN, H), lambda i: (0, 0)),
            pl.BlockSpec((1, H), lambda i: (0, 0)),
            pl.BlockSpec((H, H), lambda i: (0, 0)),
            pl.BlockSpec((H, H), lambda i: (0, 0)),
            pl.BlockSpec((H, H), lambda i: (0, 0)),
        ],
        out_specs=[
            pl.BlockSpec((TP, H), lambda i: (i, 0)),
            pl.BlockSpec((TP, 128), lambda i: (i, 0)),
        ],
        out_shape=[jax.ShapeDtypeStruct((BN, H), jnp.float32),
                   jax.ShapeDtypeStruct((BN, 128), jnp.float32)],
        compiler_params=pltpu.CompilerParams(
            dimension_semantics=("parallel",)),
    )(f2, xyz.reshape(BN, 3), W_kernel, b_kernel.reshape(1, H), W_q, W_k, W_v)

    T = 512
    xyz2 = xyz.reshape(BN, 3)
    idxs = [_knn_topk(xyz[b], b * N) for b in range(B)]
    gats = [_sc_gather(tbl, idxs[b].reshape(N * K)) for b in range(B)]

    def _mlp_batch(b, gat_b):
        nblk = N // T
        return pl.pallas_call(
            functools.partial(_mlp_body, T),
            grid=(nblk,),
            in_specs=[
                pl.BlockSpec((T, H), lambda i: (i + b * nblk, 0)),
                pl.BlockSpec((T * K, 128), lambda i: (i, 0)),
                pl.BlockSpec((T, 3), lambda i: (i + b * nblk, 0)),
                pl.BlockSpec((T, F_IN), lambda i: (i + b * nblk, 0)),
                pl.BlockSpec((3, H), lambda i: (0, 0)),
                pl.BlockSpec((1, H), lambda i: (0, 0)),
                pl.BlockSpec((H, H), lambda i: (0, 0)),
                pl.BlockSpec((1, H), lambda i: (0, 0)),
                pl.BlockSpec((H, H), lambda i: (0, 0)),
                pl.BlockSpec((1, H), lambda i: (0, 0)),
                pl.BlockSpec((H, H), lambda i: (0, 0)),
                pl.BlockSpec((1, H), lambda i: (0, 0)),
                pl.BlockSpec((H, H), lambda i: (0, 0)),
                pl.BlockSpec((1, F_IN), lambda i: (0, 0)),
            ],
            out_specs=[
                pl.BlockSpec((T * K, H), lambda i: (i, 0)),
                pl.BlockSpec((T, H), lambda i: (i, 0)),
            ],
            out_shape=[
                jax.ShapeDtypeStruct((N * K, H), jnp.float32),
                jax.ShapeDtypeStruct((N, F_IN), jnp.float32),
            ],
            compiler_params=pltpu.CompilerParams(
                dimension_semantics=("parallel",)),
        )(q, gat_b, xyz2, f2,
          pos_W1, pos_b1.reshape(1, H), pos_W2, pos_b2.reshape(1, H),
          att_W1, att_b1.reshape(1, H), att_W2, att_b2.reshape(1, H),
          W_agg, b_agg.reshape(1, F_IN))

    outs = [_mlp_batch(b, gats[b]) for b in range(B)]
    res = jnp.stack([o[1] for o in outs])
    att = jnp.stack([o[0].reshape(N, K, H) for o in outs])
    return (res, att)
